# baseline (speedup 1.0000x reference)
import functools

import jax
import jax.numpy as jnp
import numpy as np
from jax import lax
from jax.experimental import pallas as pl
from jax.experimental.pallas import tpu as pltpu
from jax.experimental.pallas import tpu_sc as plsc

_B, _Lq, _C = 4, 5376, 256
_NH, _NL, _NP = 8, 3, 4
_D = _C // _NH
_DW = _D // 2
_SHAPES = ((64, 64), (32, 32), (16, 16))
_LSI = (0, 4096, 5120)
_Lv = 5376
_NPT = _NL * _NP
_NC, _NS, _L = 2, 16, 16
_TQ = 384
_QC = 128
_NCH = _Lq // _QC

_scale_np = np.zeros((1, _NH * _NPT * 2), np.float32)
_sel_np = np.zeros((8, _NH * _NPT * 2), np.float32)
for _h in range(_NH):
    for _l in range(_NL):
        _H_, _W_ = _SHAPES[_l]
        for _p in range(_NP):
            _j = ((_h * _NL + _l) * _NP + _p) * 2
            _scale_np[0, _j] = _W_
            _scale_np[0, _j + 1] = _H_
            _sel_np[_l * 2 + 0, _j] = 1.0
            _sel_np[_l * 2 + 1, _j + 1] = 1.0
_gsum_np = np.zeros((_NH * _NPT, _NH * _NPT), np.float32)
for _i in range(_NH * _NPT):
    for _j in range(_NH * _NPT):
        if _i // _NPT == _j // _NPT:
            _gsum_np[_i, _j] = 1.0


def _prep_body(q_ref, qp_ref, ref8_ref, val_ref, woff_ref, boff_ref,
               wattn_ref, battn_ref, wv_ref, bv_ref, scale_ref, sel_ref,
               gsum_ref, xy_ref, aw_ref, vb_ref):
    q = q_ref[0] + qp_ref[0]
    off = jnp.dot(q, woff_ref[...], preferred_element_type=jnp.float32) + boff_ref[...]
    logits = jnp.dot(q, wattn_ref[...], preferred_element_type=jnp.float32) + battn_ref[...]
    m = jnp.max(logits, axis=-1, keepdims=True)
    e = jnp.exp(logits - m)
    s = jnp.dot(e, gsum_ref[...], preferred_element_type=jnp.float32)
    aw_ref[0] = jnp.transpose(e / s).reshape(_NH, _NPT, _TQ)
    ref192 = jnp.dot(ref8_ref[0], sel_ref[...], preferred_element_type=jnp.float32)
    xy = ref192 * scale_ref[...] + off - 0.5
    xy_ref[0] = jnp.transpose(xy).reshape(_NH, 2 * _NPT, _TQ)
    v = jnp.dot(val_ref[0], wv_ref[...], preferred_element_type=jnp.float32) + bv_ref[...]
    vb_ref[0] = v.astype(jnp.bfloat16)


_prep_call = pl.pallas_call(
    _prep_body,
    grid=(_B, _Lq // _TQ),
    in_specs=[
        pl.BlockSpec((1, _TQ, _C), lambda b, i: (b, i, 0)),
        pl.BlockSpec((1, _TQ, _C), lambda b, i: (b, i, 0)),
        pl.BlockSpec((1, _TQ, 8), lambda b, i: (b, i, 0)),
        pl.BlockSpec((1, _TQ, _C), lambda b, i: (b, i, 0)),
        pl.BlockSpec((_C, 192), lambda b, i: (0, 0)),
        pl.BlockSpec((1, 192), lambda b, i: (0, 0)),
        pl.BlockSpec((_C, 96), lambda b, i: (0, 0)),
        pl.BlockSpec((1, 96), lambda b, i: (0, 0)),
        pl.BlockSpec((_C, _C), lambda b, i: (0, 0)),
        pl.BlockSpec((1, _C), lambda b, i: (0, 0)),
        pl.BlockSpec((1, 192), lambda b, i: (0, 0)),
        pl.BlockSpec((8, 192), lambda b, i: (0, 0)),
        pl.BlockSpec((96, 96), lambda b, i: (0, 0)),
    ],
    out_specs=[
        pl.BlockSpec((1, _NH, 2 * _NPT, _TQ), lambda b, i: (b, 0, 0, i)),
        pl.BlockSpec((1, _NH, _NPT, _TQ), lambda b, i: (b, 0, 0, i)),
        pl.BlockSpec((1, _TQ, _C), lambda b, i: (b, i, 0)),
    ],
    out_shape=[
        jax.ShapeDtypeStruct((_B, _NH, 2 * _NPT, _Lq), jnp.float32),
        jax.ShapeDtypeStruct((_B, _NH, _NPT, _Lq), jnp.float32),
        jax.ShapeDtypeStruct((_B, _Lq, _C), jnp.bfloat16),
    ],
)


def _post_body(q_ref, attn_ref, wout_ref, bout_ref, gamma_ref, beta_ref, o_ref):
    attn = jnp.transpose(attn_ref[0].reshape(_C, _TQ))
    a = jnp.dot(attn, wout_ref[...], preferred_element_type=jnp.float32) + bout_ref[...]
    res = q_ref[0] + a
    mu = jnp.mean(res, axis=-1, keepdims=True)
    cen = res - mu
    var = jnp.mean(cen * cen, axis=-1, keepdims=True)
    o_ref[0] = cen * lax.rsqrt(var + 1e-5) * gamma_ref[...] + beta_ref[...]


def _post_call_for(t0, nq):
  return pl.pallas_call(
    _post_body,
    grid=(_B, nq // _TQ),
    in_specs=[
        pl.BlockSpec((1, _TQ, _C), lambda b, i: (b, i + t0, 0)),
        pl.BlockSpec((1, _NH, _D, _TQ), lambda b, i: (b, 0, 0, i)),
        pl.BlockSpec((_C, _C), lambda b, i: (0, 0)),
        pl.BlockSpec((1, _C), lambda b, i: (0, 0)),
        pl.BlockSpec((1, _C), lambda b, i: (0, 0)),
        pl.BlockSpec((1, _C), lambda b, i: (0, 0)),
    ],
    out_specs=pl.BlockSpec((1, _TQ, _C), lambda b, i: (b, i, 0)),
    out_shape=jax.ShapeDtypeStruct((_B, nq, _C), jnp.float32),
  )


_post_h = (_post_call_for(0, _Lq // 2), _post_call_for(_Lq // 2 // _TQ, _Lq // 2))


def _floor16(x):
    xt = x.astype(jnp.int32)
    xtf = xt.astype(jnp.float32)
    neg = x < xtf
    x0 = jnp.where(neg, xt - 1, xt)
    x0f = jnp.where(neg, xtf - 1.0, xtf)
    return x0, x0f


def _make_sample(q0, nq):
  nch = nq // _QC

  def _sample_body(tbl_hbm, xy_hbm, aw_hbm, out_hbm, tblv,
                 xyv0, xyv1, awv0, awv1, outv0, outv1,
                 xys0, xys1, aws0, aws1, outs0, outs1):
    cid = lax.axis_index("c")
    sid = lax.axis_index("s")
    wid = sid * _NC + cid
    b = wid // _NH
    h = wid % _NH

    pltpu.sync_copy(tbl_hbm.at[b, h], tblv)

    xybufs, awbufs, outbufs = (xyv0, xyv1), (awv0, awv1), (outv0, outv1)
    xysems, awsems, outsems = (xys0, xys1), (aws0, aws1), (outs0, outs1)

    def start_in(ci, k):
        qs = pl.ds(q0 + ci * _QC, _QC)
        pltpu.async_copy(xy_hbm.at[b, h, :, qs], xybufs[k], xysems[k])
        pltpu.async_copy(aw_hbm.at[b, h, :, qs], awbufs[k], awsems[k])

    def wait_in(k):
        qs = pl.ds(0, _QC)
        pltpu.make_async_copy(xy_hbm.at[b, h, :, qs], xybufs[k], xysems[k]).wait()
        pltpu.make_async_copy(aw_hbm.at[b, h, :, qs], awbufs[k], awsems[k]).wait()

    def start_out(ci, k):
        qs = pl.ds(ci * _QC, _QC)
        pltpu.async_copy(outbufs[k], out_hbm.at[b, h, :, qs], outsems[k])

    def wait_out(k):
        qs = pl.ds(0, _QC)
        pltpu.make_async_copy(outbufs[k], out_hbm.at[b, h, :, qs], outsems[k]).wait()

    def make_group_body(xyv, awv, outv):
      def group_body(g, carry):
        qb = g * _L
        acc = [jnp.zeros((2 * _L,), jnp.bfloat16) for _ in range(_DW)]
        for pt in range(_NPT):
            lvl = pt // _NP
            h_l, w_l = _SHAPES[lvl]
            base = _LSI[lvl]
            xv = xyv[2 * pt, pl.ds(qb, _L)]
            yv = xyv[2 * pt + 1, pl.ds(qb, _L)]
            av = awv[pt, pl.ds(qb, _L)]
            x0, x0f = _floor16(xv)
            y0, y0f = _floor16(yv)
            x1 = x0 + 1
            y1 = y0 + 1
            fx1 = xv - x0f
            fx0 = 1.0 - fx1
            fy1 = yv - y0f
            fy0 = 1.0 - fy1
            wx0 = jnp.where((x0 >= 0) & (x0 < w_l), fx0, 0.0)
            wx1 = jnp.where((x1 >= 0) & (x1 < w_l), fx1, 0.0)
            wy0 = jnp.where((y0 >= 0) & (y0 < h_l), fy0 * av, 0.0)
            wy1 = jnp.where((y1 >= 0) & (y1 < h_l), fy1 * av, 0.0)
            xc0 = jnp.clip(x0, 0, w_l - 1)
            xc1 = jnp.clip(x1, 0, w_l - 1)
            yc0 = jnp.clip(y0, 0, h_l - 1) * w_l
            yc1 = jnp.clip(y1, 0, h_l - 1) * w_l
            for yci, xci, wgt in ((yc0, xc0, wy0 * wx0), (yc0, xc1, wy0 * wx1),
                                  (yc1, xc0, wy1 * wx0), (yc1, xc1, wy1 * wx1)):
                wb = plsc.pack(wgt, wgt, format=plsc.PackFormat.INTERLEAVED)
                rowv = base + yci + xci
                for dw in range(_DW):
                    wv = plsc.load_gather(tblv, [rowv + (dw * _Lv)])
                    vb = plsc.bitcast(wv, jnp.bfloat16)
                    acc[dw] = acc[dw] + vb * wb
        for dw in range(_DW):
            a0, a1 = plsc.unpack(acc[dw], format=plsc.PackFormat.INTERLEAVED)
            outv[2 * dw, pl.ds(qb, _L)] = a0
            outv[2 * dw + 1, pl.ds(qb, _L)] = a1
        return carry
      return group_body

    group_bodies = (make_group_body(xyv0, awv0, outv0),
                    make_group_body(xyv1, awv1, outv1))

    start_in(0, 0)

    def pair_body(i, carry):
        for k in (0, 1):
            ci = 2 * i + k

            @pl.when(ci + 1 < nch)
            def _():
                start_in(ci + 1, 1 - k)

            wait_in(k)

            @pl.when(ci >= 2)
            def _():
                wait_out(k)

            lax.fori_loop(0, _QC // _L, group_bodies[k], 0)
            start_out(ci, k)
        return carry

    lax.fori_loop(0, nch // 2, pair_body, 0)
    if nch % 2:
        ci = nch - 1
        k = ci % 2
        wait_in(k)
        wait_out(k)
        lax.fori_loop(0, _QC // _L, group_bodies[k], 0)
        start_out(ci, k)
    wait_out(0)
    wait_out(1)

  return _sample_body


def _sample_call_for(q0, nq):
  return functools.partial(
    pl.kernel,
    out_type=jax.ShapeDtypeStruct((_B, _NH, _D, nq), jnp.float32),
    mesh=plsc.VectorSubcoreMesh(core_axis_name="c", subcore_axis_name="s",
                                num_cores=_NC, num_subcores=_NS),
    compiler_params=pltpu.CompilerParams(needs_layout_passes=False),
    scratch_types=[
        pltpu.VMEM((_DW * _Lv,), jnp.int32),
        pltpu.VMEM((2 * _NPT, _QC), jnp.float32),
        pltpu.VMEM((2 * _NPT, _QC), jnp.float32),
        pltpu.VMEM((_NPT, _QC), jnp.float32),
        pltpu.VMEM((_NPT, _QC), jnp.float32),
        pltpu.VMEM((_D, _QC), jnp.float32),
        pltpu.VMEM((_D, _QC), jnp.float32),
        pltpu.SemaphoreType.DMA,
        pltpu.SemaphoreType.DMA,
        pltpu.SemaphoreType.DMA,
        pltpu.SemaphoreType.DMA,
        pltpu.SemaphoreType.DMA,
        pltpu.SemaphoreType.DMA,
    ],
  )(_make_sample(q0, nq))


_HALF = _Lq // 2
_sample_h = (_sample_call_for(0, _HALF), _sample_call_for(_HALF, _HALF))


def kernel(query, value, query_pos, ref_pts, spatial_shapes, level_start_index,
           W_off, b_off, W_attn, b_attn, W_v, b_v, W_out, b_out, gamma, beta):
    del spatial_shapes, level_start_index
    ref8 = jnp.pad(ref_pts.reshape(_B, _Lq, _NL * 2), ((0, 0), (0, 0), (0, 2)))
    xy, aw, vb = _prep_call(
        query, query_pos, ref8, value,
        W_off, b_off.reshape(1, -1), W_attn, b_attn.reshape(1, -1),
        W_v, b_v.reshape(1, -1),
        jnp.asarray(_scale_np), jnp.asarray(_sel_np), jnp.asarray(_gsum_np))
    vi = lax.bitcast_convert_type(
        vb.reshape(_B, _Lv, _NH, _DW, 2), jnp.int32)
    tbl = jnp.transpose(vi, (0, 2, 3, 1)).reshape(_B, _NH, _DW * _Lv)
    o1 = _sample_h[0](tbl, xy, aw)
    o2 = _sample_h[1](tbl, xy, aw)
    r1 = _post_h[0](query, o1, W_out, b_out.reshape(1, -1),
                    gamma.reshape(1, -1), beta.reshape(1, -1))
    r2 = _post_h[1](query, o2, W_out, b_out.reshape(1, -1),
                    gamma.reshape(1, -1), beta.reshape(1, -1))
    return jnp.concatenate([r1, r2], axis=1)


# speedup vs baseline: 1.0346x; 1.0346x over previous
import functools

import jax
import jax.numpy as jnp
import numpy as np
from jax import lax
from jax.experimental import pallas as pl
from jax.experimental.pallas import tpu as pltpu
from jax.experimental.pallas import tpu_sc as plsc

_B, _Lq, _C = 4, 5376, 256
_NH, _NL, _NP = 8, 3, 4
_D = _C // _NH
_DW = _D // 2
_SHAPES = ((64, 64), (32, 32), (16, 16))
_LSI = (0, 4096, 5120)
_Lv = 5376
_NPT = _NL * _NP
_NC, _NS, _L = 2, 16, 16
_TQ = 384
_QC = 256
_NCH = _Lq // _QC

_scale_np = np.zeros((1, _NH * _NPT * 2), np.float32)
_sel_np = np.zeros((8, _NH * _NPT * 2), np.float32)
for _h in range(_NH):
    for _l in range(_NL):
        _H_, _W_ = _SHAPES[_l]
        for _p in range(_NP):
            _j = ((_h * _NL + _l) * _NP + _p) * 2
            _scale_np[0, _j] = _W_
            _scale_np[0, _j + 1] = _H_
            _sel_np[_l * 2 + 0, _j] = 1.0
            _sel_np[_l * 2 + 1, _j + 1] = 1.0
_gsum_np = np.zeros((_NH * _NPT, _NH * _NPT), np.float32)
for _i in range(_NH * _NPT):
    for _j in range(_NH * _NPT):
        if _i // _NPT == _j // _NPT:
            _gsum_np[_i, _j] = 1.0


def _prep_body(q_ref, qp_ref, ref8_ref, val_ref, woff_ref, boff_ref,
               wattn_ref, battn_ref, wv_ref, bv_ref, scale_ref, sel_ref,
               gsum_ref, xy_ref, aw_ref, vb_ref):
    q = q_ref[0] + qp_ref[0]
    off = jnp.dot(q, woff_ref[...], preferred_element_type=jnp.float32) + boff_ref[...]
    logits = jnp.dot(q, wattn_ref[...], preferred_element_type=jnp.float32) + battn_ref[...]
    m = jnp.max(logits, axis=-1, keepdims=True)
    e = jnp.exp(logits - m)
    s = jnp.dot(e, gsum_ref[...], preferred_element_type=jnp.float32)
    aw_ref[0] = jnp.transpose(e / s).reshape(_NH, _NPT, _TQ)
    ref192 = jnp.dot(ref8_ref[0], sel_ref[...], preferred_element_type=jnp.float32)
    xy = ref192 * scale_ref[...] + off - 0.5
    xy_ref[0] = jnp.transpose(xy).reshape(_NH, 2 * _NPT, _TQ)
    v = jnp.dot(val_ref[0], wv_ref[...], preferred_element_type=jnp.float32) + bv_ref[...]
    vb_ref[0] = v.astype(jnp.bfloat16)


_prep_call = pl.pallas_call(
    _prep_body,
    grid=(_B, _Lq // _TQ),
    in_specs=[
        pl.BlockSpec((1, _TQ, _C), lambda b, i: (b, i, 0)),
        pl.BlockSpec((1, _TQ, _C), lambda b, i: (b, i, 0)),
        pl.BlockSpec((1, _TQ, 8), lambda b, i: (b, i, 0)),
        pl.BlockSpec((1, _TQ, _C), lambda b, i: (b, i, 0)),
        pl.BlockSpec((_C, 192), lambda b, i: (0, 0)),
        pl.BlockSpec((1, 192), lambda b, i: (0, 0)),
        pl.BlockSpec((_C, 96), lambda b, i: (0, 0)),
        pl.BlockSpec((1, 96), lambda b, i: (0, 0)),
        pl.BlockSpec((_C, _C), lambda b, i: (0, 0)),
        pl.BlockSpec((1, _C), lambda b, i: (0, 0)),
        pl.BlockSpec((1, 192), lambda b, i: (0, 0)),
        pl.BlockSpec((8, 192), lambda b, i: (0, 0)),
        pl.BlockSpec((96, 96), lambda b, i: (0, 0)),
    ],
    out_specs=[
        pl.BlockSpec((1, _NH, 2 * _NPT, _TQ), lambda b, i: (b, 0, 0, i)),
        pl.BlockSpec((1, _NH, _NPT, _TQ), lambda b, i: (b, 0, 0, i)),
        pl.BlockSpec((1, _TQ, _C), lambda b, i: (b, i, 0)),
    ],
    out_shape=[
        jax.ShapeDtypeStruct((_B, _NH, 2 * _NPT, _Lq), jnp.float32),
        jax.ShapeDtypeStruct((_B, _NH, _NPT, _Lq), jnp.float32),
        jax.ShapeDtypeStruct((_B, _Lq, _C), jnp.bfloat16),
    ],
)


def _post_body(q_ref, attn_ref, wout_ref, bout_ref, gamma_ref, beta_ref, o_ref):
    attn = jnp.transpose(attn_ref[0].reshape(_C, _TQ))
    a = jnp.dot(attn, wout_ref[...], preferred_element_type=jnp.float32) + bout_ref[...]
    res = q_ref[0] + a
    mu = jnp.mean(res, axis=-1, keepdims=True)
    cen = res - mu
    var = jnp.mean(cen * cen, axis=-1, keepdims=True)
    o_ref[0] = cen * lax.rsqrt(var + 1e-5) * gamma_ref[...] + beta_ref[...]


def _post_call_for(t0, nq):
  return pl.pallas_call(
    _post_body,
    grid=(_B, nq // _TQ),
    in_specs=[
        pl.BlockSpec((1, _TQ, _C), lambda b, i: (b, i + t0, 0)),
        pl.BlockSpec((1, _NH, _D, _TQ), lambda b, i: (b, 0, 0, i)),
        pl.BlockSpec((_C, _C), lambda b, i: (0, 0)),
        pl.BlockSpec((1, _C), lambda b, i: (0, 0)),
        pl.BlockSpec((1, _C), lambda b, i: (0, 0)),
        pl.BlockSpec((1, _C), lambda b, i: (0, 0)),
    ],
    out_specs=pl.BlockSpec((1, _TQ, _C), lambda b, i: (b, i, 0)),
    out_shape=jax.ShapeDtypeStruct((_B, nq, _C), jnp.float32),
  )


_post_one = _post_call_for(0, _Lq)


def _floor16(x):
    xt = x.astype(jnp.int32)
    xtf = xt.astype(jnp.float32)
    neg = x < xtf
    x0 = jnp.where(neg, xt - 1, xt)
    x0f = jnp.where(neg, xtf - 1.0, xtf)
    return x0, x0f


def _make_sample(q0, nq):
  nch = nq // _QC

  def _sample_body(tbl_hbm, xy_hbm, aw_hbm, out_hbm, tblv,
                 xyv0, xyv1, awv0, awv1, outv0, outv1,
                 xys0, xys1, aws0, aws1, outs0, outs1):
    cid = lax.axis_index("c")
    sid = lax.axis_index("s")
    wid = sid * _NC + cid
    b = wid // _NH
    h = wid % _NH

    pltpu.sync_copy(tbl_hbm.at[b, h], tblv)

    xybufs, awbufs, outbufs = (xyv0, xyv1), (awv0, awv1), (outv0, outv1)
    xysems, awsems, outsems = (xys0, xys1), (aws0, aws1), (outs0, outs1)

    def start_in(ci, k):
        qs = pl.ds(q0 + ci * _QC, _QC)
        pltpu.async_copy(xy_hbm.at[b, h, :, qs], xybufs[k], xysems[k])
        pltpu.async_copy(aw_hbm.at[b, h, :, qs], awbufs[k], awsems[k])

    def wait_in(k):
        qs = pl.ds(0, _QC)
        pltpu.make_async_copy(xy_hbm.at[b, h, :, qs], xybufs[k], xysems[k]).wait()
        pltpu.make_async_copy(aw_hbm.at[b, h, :, qs], awbufs[k], awsems[k]).wait()

    def start_out(ci, k):
        qs = pl.ds(ci * _QC, _QC)
        pltpu.async_copy(outbufs[k], out_hbm.at[b, h, :, qs], outsems[k])

    def wait_out(k):
        qs = pl.ds(0, _QC)
        pltpu.make_async_copy(outbufs[k], out_hbm.at[b, h, :, qs], outsems[k]).wait()

    def make_group_body(xyv, awv, outv):
      def group_body(g, carry):
        qb = g * _L
        acc = [jnp.zeros((2 * _L,), jnp.bfloat16) for _ in range(_DW)]
        for pt in range(_NPT):
            lvl = pt // _NP
            h_l, w_l = _SHAPES[lvl]
            base = _LSI[lvl]
            xv = xyv[2 * pt, pl.ds(qb, _L)]
            yv = xyv[2 * pt + 1, pl.ds(qb, _L)]
            av = awv[pt, pl.ds(qb, _L)]
            x0, x0f = _floor16(xv)
            y0, y0f = _floor16(yv)
            x1 = x0 + 1
            y1 = y0 + 1
            fx1 = xv - x0f
            fx0 = 1.0 - fx1
            fy1 = yv - y0f
            fy0 = 1.0 - fy1
            wx0 = jnp.where((x0 >= 0) & (x0 < w_l), fx0, 0.0)
            wx1 = jnp.where((x1 >= 0) & (x1 < w_l), fx1, 0.0)
            wy0 = jnp.where((y0 >= 0) & (y0 < h_l), fy0 * av, 0.0)
            wy1 = jnp.where((y1 >= 0) & (y1 < h_l), fy1 * av, 0.0)
            xc0 = jnp.clip(x0, 0, w_l - 1)
            xc1 = jnp.clip(x1, 0, w_l - 1)
            yc0 = jnp.clip(y0, 0, h_l - 1) * w_l
            yc1 = jnp.clip(y1, 0, h_l - 1) * w_l
            for yci, xci, wgt in ((yc0, xc0, wy0 * wx0), (yc0, xc1, wy0 * wx1),
                                  (yc1, xc0, wy1 * wx0), (yc1, xc1, wy1 * wx1)):
                wb = plsc.pack(wgt, wgt, format=plsc.PackFormat.INTERLEAVED)
                rowv = base + yci + xci
                for dw in range(_DW):
                    wv = plsc.load_gather(tblv, [rowv + (dw * _Lv)])
                    vb = plsc.bitcast(wv, jnp.bfloat16)
                    acc[dw] = acc[dw] + vb * wb
        for dw in range(_DW):
            a0, a1 = plsc.unpack(acc[dw], format=plsc.PackFormat.INTERLEAVED)
            outv[2 * dw, pl.ds(qb, _L)] = a0
            outv[2 * dw + 1, pl.ds(qb, _L)] = a1
        return carry
      return group_body

    group_bodies = (make_group_body(xyv0, awv0, outv0),
                    make_group_body(xyv1, awv1, outv1))

    start_in(0, 0)

    def pair_body(i, carry):
        for k in (0, 1):
            ci = 2 * i + k

            @pl.when(ci + 1 < nch)
            def _():
                start_in(ci + 1, 1 - k)

            wait_in(k)

            @pl.when(ci >= 2)
            def _():
                wait_out(k)

            lax.fori_loop(0, _QC // _L, group_bodies[k], 0)
            start_out(ci, k)
        return carry

    lax.fori_loop(0, nch // 2, pair_body, 0)
    if nch % 2:
        ci = nch - 1
        k = ci % 2
        wait_in(k)
        wait_out(k)
        lax.fori_loop(0, _QC // _L, group_bodies[k], 0)
        start_out(ci, k)
    wait_out(0)
    wait_out(1)

  return _sample_body


def _sample_call_for(q0, nq):
  return functools.partial(
    pl.kernel,
    out_type=jax.ShapeDtypeStruct((_B, _NH, _D, nq), jnp.float32),
    mesh=plsc.VectorSubcoreMesh(core_axis_name="c", subcore_axis_name="s",
                                num_cores=_NC, num_subcores=_NS),
    compiler_params=pltpu.CompilerParams(needs_layout_passes=False),
    scratch_types=[
        pltpu.VMEM((_DW * _Lv,), jnp.int32),
        pltpu.VMEM((2 * _NPT, _QC), jnp.float32),
        pltpu.VMEM((2 * _NPT, _QC), jnp.float32),
        pltpu.VMEM((_NPT, _QC), jnp.float32),
        pltpu.VMEM((_NPT, _QC), jnp.float32),
        pltpu.VMEM((_D, _QC), jnp.float32),
        pltpu.VMEM((_D, _QC), jnp.float32),
        pltpu.SemaphoreType.DMA,
        pltpu.SemaphoreType.DMA,
        pltpu.SemaphoreType.DMA,
        pltpu.SemaphoreType.DMA,
        pltpu.SemaphoreType.DMA,
        pltpu.SemaphoreType.DMA,
    ],
  )(_make_sample(q0, nq))


_sample_one = _sample_call_for(0, _Lq)


def kernel(query, value, query_pos, ref_pts, spatial_shapes, level_start_index,
           W_off, b_off, W_attn, b_attn, W_v, b_v, W_out, b_out, gamma, beta):
    del spatial_shapes, level_start_index
    ref8 = jnp.pad(ref_pts.reshape(_B, _Lq, _NL * 2), ((0, 0), (0, 0), (0, 2)))
    xy, aw, vb = _prep_call(
        query, query_pos, ref8, value,
        W_off, b_off.reshape(1, -1), W_attn, b_attn.reshape(1, -1),
        W_v, b_v.reshape(1, -1),
        jnp.asarray(_scale_np), jnp.asarray(_sel_np), jnp.asarray(_gsum_np))
    vi = lax.bitcast_convert_type(
        vb.reshape(_B, _Lv, _NH, _DW, 2), jnp.int32)
    tbl = jnp.transpose(vi, (0, 2, 3, 1)).reshape(_B, _NH, _DW * _Lv)
    out_t = _sample_one(tbl, xy, aw)
    return _post_one(query, out_t, W_out, b_out.reshape(1, -1),
                     gamma.reshape(1, -1), beta.reshape(1, -1))


# speedup vs baseline: 1.0359x; 1.0012x over previous
import functools

import jax
import jax.numpy as jnp
import numpy as np
from jax import lax
from jax.experimental import pallas as pl
from jax.experimental.pallas import tpu as pltpu
from jax.experimental.pallas import tpu_sc as plsc

_B, _Lq, _C = 4, 5376, 256
_NH, _NL, _NP = 8, 3, 4
_D = _C // _NH
_DW = _D // 2
_SHAPES = ((64, 64), (32, 32), (16, 16))
_LSI = (0, 4096, 5120)
_Lv = 5376
_NPT = _NL * _NP
_NC, _NS, _L = 2, 16, 16
_TQ = 384
_QC = 128
_NCH = _Lq // _QC

_scale_np = np.zeros((1, _NH * _NPT * 2), np.float32)
_sel_np = np.zeros((8, _NH * _NPT * 2), np.float32)
for _h in range(_NH):
    for _l in range(_NL):
        _H_, _W_ = _SHAPES[_l]
        for _p in range(_NP):
            _j = ((_h * _NL + _l) * _NP + _p) * 2
            _scale_np[0, _j] = _W_
            _scale_np[0, _j + 1] = _H_
            _sel_np[_l * 2 + 0, _j] = 1.0
            _sel_np[_l * 2 + 1, _j + 1] = 1.0
_gsum_np = np.zeros((_NH * _NPT, _NH * _NPT), np.float32)
for _i in range(_NH * _NPT):
    for _j in range(_NH * _NPT):
        if _i // _NPT == _j // _NPT:
            _gsum_np[_i, _j] = 1.0


def _prep_body(q_ref, qp_ref, ref8_ref, val_ref, woff_ref, boff_ref,
               wattn_ref, battn_ref, wv_ref, bv_ref, scale_ref, sel_ref,
               gsum_ref, xy_ref, aw_ref, vb_ref):
    q = q_ref[0] + qp_ref[0]
    off = jnp.dot(q, woff_ref[...], preferred_element_type=jnp.float32) + boff_ref[...]
    logits = jnp.dot(q, wattn_ref[...], preferred_element_type=jnp.float32) + battn_ref[...]
    m = jnp.max(logits, axis=-1, keepdims=True)
    e = jnp.exp(logits - m)
    s = jnp.dot(e, gsum_ref[...], preferred_element_type=jnp.float32)
    aw_ref[0] = jnp.transpose(e / s).reshape(_NH, _NPT, _TQ)
    ref192 = jnp.dot(ref8_ref[0], sel_ref[...], preferred_element_type=jnp.float32)
    xy = ref192 * scale_ref[...] + off - 0.5
    xy_ref[0] = jnp.transpose(xy).reshape(_NH, 2 * _NPT, _TQ)
    v = jnp.dot(val_ref[0], wv_ref[...], preferred_element_type=jnp.float32) + bv_ref[...]
    vb_ref[0] = v.astype(jnp.bfloat16)


_prep_call = pl.pallas_call(
    _prep_body,
    grid=(_B, _Lq // _TQ),
    in_specs=[
        pl.BlockSpec((1, _TQ, _C), lambda b, i: (b, i, 0)),
        pl.BlockSpec((1, _TQ, _C), lambda b, i: (b, i, 0)),
        pl.BlockSpec((1, _TQ, 8), lambda b, i: (b, i, 0)),
        pl.BlockSpec((1, _TQ, _C), lambda b, i: (b, i, 0)),
        pl.BlockSpec((_C, 192), lambda b, i: (0, 0)),
        pl.BlockSpec((1, 192), lambda b, i: (0, 0)),
        pl.BlockSpec((_C, 96), lambda b, i: (0, 0)),
        pl.BlockSpec((1, 96), lambda b, i: (0, 0)),
        pl.BlockSpec((_C, _C), lambda b, i: (0, 0)),
        pl.BlockSpec((1, _C), lambda b, i: (0, 0)),
        pl.BlockSpec((1, 192), lambda b, i: (0, 0)),
        pl.BlockSpec((8, 192), lambda b, i: (0, 0)),
        pl.BlockSpec((96, 96), lambda b, i: (0, 0)),
    ],
    out_specs=[
        pl.BlockSpec((1, _NH, 2 * _NPT, _TQ), lambda b, i: (b, 0, 0, i)),
        pl.BlockSpec((1, _NH, _NPT, _TQ), lambda b, i: (b, 0, 0, i)),
        pl.BlockSpec((1, _TQ, _C), lambda b, i: (b, i, 0)),
    ],
    out_shape=[
        jax.ShapeDtypeStruct((_B, _NH, 2 * _NPT, _Lq), jnp.float32),
        jax.ShapeDtypeStruct((_B, _NH, _NPT, _Lq), jnp.float32),
        jax.ShapeDtypeStruct((_B, _Lq, _C), jnp.bfloat16),
    ],
)


def _post_body(q_ref, attn_ref, wout_ref, bout_ref, gamma_ref, beta_ref, o_ref):
    attn = jnp.transpose(attn_ref[0].reshape(_C, _TQ))
    a = jnp.dot(attn, wout_ref[...], preferred_element_type=jnp.float32) + bout_ref[...]
    res = q_ref[0] + a
    mu = jnp.mean(res, axis=-1, keepdims=True)
    cen = res - mu
    var = jnp.mean(cen * cen, axis=-1, keepdims=True)
    o_ref[0] = cen * lax.rsqrt(var + 1e-5) * gamma_ref[...] + beta_ref[...]


_post_call = pl.pallas_call(
    _post_body,
    grid=(_B, _Lq // _TQ),
    in_specs=[
        pl.BlockSpec((1, _TQ, _C), lambda b, i: (b, i, 0)),
        pl.BlockSpec((1, _NH, _D, _TQ), lambda b, i: (b, 0, 0, i)),
        pl.BlockSpec((_C, _C), lambda b, i: (0, 0)),
        pl.BlockSpec((1, _C), lambda b, i: (0, 0)),
        pl.BlockSpec((1, _C), lambda b, i: (0, 0)),
        pl.BlockSpec((1, _C), lambda b, i: (0, 0)),
    ],
    out_specs=pl.BlockSpec((1, _TQ, _C), lambda b, i: (b, i, 0)),
    out_shape=jax.ShapeDtypeStruct((_B, _Lq, _C), jnp.float32),
)


def _floor16(x):
    xt = x.astype(jnp.int32)
    xtf = xt.astype(jnp.float32)
    neg = x < xtf
    x0 = jnp.where(neg, xt - 1, xt)
    x0f = jnp.where(neg, xtf - 1.0, xtf)
    return x0, x0f


def _sample_body(tbl_hbm, xy_hbm, aw_hbm, out_hbm, tblv,
                 xyv0, xyv1, awv0, awv1, outv0, outv1,
                 xys0, xys1, aws0, aws1, outs0, outs1):
    cid = lax.axis_index("c")
    sid = lax.axis_index("s")
    wid = sid * _NC + cid
    b = wid // _NH
    h = wid % _NH

    pltpu.sync_copy(tbl_hbm.at[b, h], tblv)

    xybufs, awbufs, outbufs = (xyv0, xyv1), (awv0, awv1), (outv0, outv1)
    xysems, awsems, outsems = (xys0, xys1), (aws0, aws1), (outs0, outs1)

    def start_in(ci, k):
        qs = pl.ds(ci * _QC, _QC)
        pltpu.async_copy(xy_hbm.at[b, h, :, qs], xybufs[k], xysems[k])
        pltpu.async_copy(aw_hbm.at[b, h, :, qs], awbufs[k], awsems[k])

    def wait_in(k):
        qs = pl.ds(0, _QC)
        pltpu.make_async_copy(xy_hbm.at[b, h, :, qs], xybufs[k], xysems[k]).wait()
        pltpu.make_async_copy(aw_hbm.at[b, h, :, qs], awbufs[k], awsems[k]).wait()

    def start_out(ci, k):
        qs = pl.ds(ci * _QC, _QC)
        pltpu.async_copy(outbufs[k], out_hbm.at[b, h, :, qs], outsems[k])

    def wait_out(k):
        qs = pl.ds(0, _QC)
        pltpu.make_async_copy(outbufs[k], out_hbm.at[b, h, :, qs], outsems[k]).wait()

    def make_group_body(xyv, awv, outv):
      def group_body(g, carry):
        qb = g * _L
        acc = [jnp.zeros((2 * _L,), jnp.bfloat16) for _ in range(_DW)]
        for pt in range(_NPT):
            lvl = pt // _NP
            h_l, w_l = _SHAPES[lvl]
            base = _LSI[lvl]
            xv = xyv[2 * pt, pl.ds(qb, _L)]
            yv = xyv[2 * pt + 1, pl.ds(qb, _L)]
            av = awv[pt, pl.ds(qb, _L)]
            x0, x0f = _floor16(xv)
            y0, y0f = _floor16(yv)
            x1 = x0 + 1
            y1 = y0 + 1
            fx1 = xv - x0f
            fx0 = 1.0 - fx1
            fy1 = yv - y0f
            fy0 = 1.0 - fy1
            wx0 = jnp.where((x0 >= 0) & (x0 < w_l), fx0, 0.0)
            wx1 = jnp.where((x1 >= 0) & (x1 < w_l), fx1, 0.0)
            wy0 = jnp.where((y0 >= 0) & (y0 < h_l), fy0 * av, 0.0)
            wy1 = jnp.where((y1 >= 0) & (y1 < h_l), fy1 * av, 0.0)
            xc0 = jnp.clip(x0, 0, w_l - 1)
            xc1 = jnp.clip(x1, 0, w_l - 1)
            yc0 = jnp.clip(y0, 0, h_l - 1) * w_l
            yc1 = jnp.clip(y1, 0, h_l - 1) * w_l
            for yci, xci, wgt in ((yc0, xc0, wy0 * wx0), (yc0, xc1, wy0 * wx1),
                                  (yc1, xc0, wy1 * wx0), (yc1, xc1, wy1 * wx1)):
                wb = plsc.pack(wgt, wgt, format=plsc.PackFormat.INTERLEAVED)
                rowv = base + yci + xci
                for dw in range(_DW):
                    wv = plsc.load_gather(tblv, [rowv + (dw * _Lv)])
                    vb = plsc.bitcast(wv, jnp.bfloat16)
                    acc[dw] = acc[dw] + vb * wb
        for dw in range(_DW):
            a0, a1 = plsc.unpack(acc[dw], format=plsc.PackFormat.INTERLEAVED)
            outv[2 * dw, pl.ds(qb, _L)] = a0
            outv[2 * dw + 1, pl.ds(qb, _L)] = a1
        return carry
      return group_body

    group_bodies = (make_group_body(xyv0, awv0, outv0),
                    make_group_body(xyv1, awv1, outv1))

    start_in(0, 0)

    def pair_body(i, carry):
        for k in (0, 1):
            ci = 2 * i + k

            @pl.when(ci + 1 < _NCH)
            def _():
                start_in(ci + 1, 1 - k)

            wait_in(k)

            @pl.when(ci >= 2)
            def _():
                wait_out(k)

            lax.fori_loop(0, _QC // _L, group_bodies[k], 0)
            start_out(ci, k)
        return carry

    lax.fori_loop(0, _NCH // 2, pair_body, 0)
    wait_out(0)
    wait_out(1)


_sample_call = functools.partial(
    pl.kernel,
    out_type=jax.ShapeDtypeStruct((_B, _NH, _D, _Lq), jnp.float32),
    mesh=plsc.VectorSubcoreMesh(core_axis_name="c", subcore_axis_name="s",
                                num_cores=_NC, num_subcores=_NS),
    compiler_params=pltpu.CompilerParams(needs_layout_passes=False),
    scratch_types=[
        pltpu.VMEM((_DW * _Lv,), jnp.int32),
        pltpu.VMEM((2 * _NPT, _QC), jnp.float32),
        pltpu.VMEM((2 * _NPT, _QC), jnp.float32),
        pltpu.VMEM((_NPT, _QC), jnp.float32),
        pltpu.VMEM((_NPT, _QC), jnp.float32),
        pltpu.VMEM((_D, _QC), jnp.float32),
        pltpu.VMEM((_D, _QC), jnp.float32),
        pltpu.SemaphoreType.DMA,
        pltpu.SemaphoreType.DMA,
        pltpu.SemaphoreType.DMA,
        pltpu.SemaphoreType.DMA,
        pltpu.SemaphoreType.DMA,
        pltpu.SemaphoreType.DMA,
    ],
)(_sample_body)


def kernel(query, value, query_pos, ref_pts, spatial_shapes, level_start_index,
           W_off, b_off, W_attn, b_attn, W_v, b_v, W_out, b_out, gamma, beta):
    del spatial_shapes, level_start_index
    ref8 = jnp.pad(ref_pts.reshape(_B, _Lq, _NL * 2), ((0, 0), (0, 0), (0, 2)))
    xy, aw, vb = _prep_call(
        query, query_pos, ref8, value,
        W_off, b_off.reshape(1, -1), W_attn, b_attn.reshape(1, -1),
        W_v, b_v.reshape(1, -1),
        jnp.asarray(_scale_np), jnp.asarray(_sel_np), jnp.asarray(_gsum_np))
    vi = lax.bitcast_convert_type(
        vb.reshape(_B, _Lv, _NH, _DW, 2), jnp.int32)
    tbl = jnp.transpose(vi, (0, 2, 3, 1)).reshape(_B, _NH, _DW * _Lv)
    out_t = _sample_call(tbl, xy, aw)
    return _post_call(query, out_t, W_out, b_out.reshape(1, -1),
                      gamma.reshape(1, -1), beta.reshape(1, -1))
